# initial kernel scaffold (unmeasured)
import jax
import jax.numpy as jnp
from jax import lax
from jax.experimental import pallas as pl
from jax.experimental.pallas import tpu as pltpu


def kernel(
    x,
):
    def body(*refs):
        pass

    out_shape = jax.ShapeDtypeStruct(..., jnp.float32)
    return pl.pallas_call(body, out_shape=out_shape)(...)


# baseline (device time: 352425 ns/iter reference)
import functools

import jax
import jax.numpy as jnp
from jax import lax
from jax.experimental import pallas as pl
from jax.experimental.pallas import tpu as pltpu

N_DEV = 8


def kernel(x):
    _, m, n_total = x.shape
    n_per = n_total // N_DEV

    def body(x_ref, out_ref, send_ref, recv_ref, send_sems, recv_sems, credit_sem):
        my = lax.axis_index("i")
        left = lax.rem(my + N_DEV - 1, N_DEV)
        right = lax.rem(my + 1, N_DEV)

        barrier_sem = pltpu.get_barrier_semaphore()
        for nbr in (left, right):
            pl.semaphore_signal(
                barrier_sem, inc=1,
                device_id=(nbr,), device_id_type=pl.DeviceIdType.MESH,
            )
        pl.semaphore_wait(barrier_sem, 2)

        def chunk(c):
            return x_ref[0, :, pl.ds(c * n_per, n_per)]

        send_ref[...] = chunk(left)

        for h in range(N_DEV - 1):
            slot = (h + 1) % 2
            if h >= 2:
                pl.semaphore_wait(credit_sem, 1)
            rdma = pltpu.make_async_remote_copy(
                src_ref=send_ref,
                dst_ref=recv_ref.at[slot],
                send_sem=send_sems.at[slot],
                recv_sem=recv_sems.at[slot],
                device_id=(right,),
                device_id_type=pl.DeviceIdType.MESH,
            )
            rdma.start()
            rdma.wait()

            c = lax.rem(my + 2 * N_DEV - h - 2, N_DEV)
            if h < N_DEV - 2:
                send_ref[...] = recv_ref[slot] + chunk(c)
            else:
                out_ref[...] = recv_ref[slot] + chunk(c)
            if h <= N_DEV - 4:
                pl.semaphore_signal(
                    credit_sem, inc=1,
                    device_id=(left,), device_id_type=pl.DeviceIdType.MESH,
                )

        @functools.partial(
            pl.run_scoped, second_barrier=pltpu.SemaphoreType.REGULAR
        )
        def _(second_barrier):
            for nbr in (left, right):
                pl.semaphore_signal(
                    second_barrier, inc=1,
                    device_id=(nbr,), device_id_type=pl.DeviceIdType.MESH,
                )
            pl.semaphore_wait(second_barrier, 2)

    return pl.pallas_call(
        body,
        out_shape=jax.ShapeDtypeStruct((m, n_per), x.dtype),
        in_specs=[pl.BlockSpec(memory_space=pltpu.VMEM)],
        out_specs=pl.BlockSpec(memory_space=pltpu.VMEM),
        scratch_shapes=[
            pltpu.VMEM((m, n_per), x.dtype),
            pltpu.VMEM((2, m, n_per), x.dtype),
            pltpu.SemaphoreType.DMA((2,)),
            pltpu.SemaphoreType.DMA((2,)),
            pltpu.SemaphoreType.REGULAR,
        ],
        compiler_params=pltpu.CompilerParams(
            collective_id=0,
            vmem_limit_bytes=60 * 1024 * 1024,
        ),
    )(x)


# device time: 198080 ns/iter; 1.7792x vs baseline; 1.7792x over previous
import functools

import jax
import jax.numpy as jnp
from jax import lax
from jax.experimental import pallas as pl
from jax.experimental.pallas import tpu as pltpu

N_DEV = 8


def kernel(x):
    _, m, n_total = x.shape
    n_per = n_total // N_DEV
    m_half = m // 2

    def body(
        x_ref,
        out_ref,
        send_f,
        send_b,
        recv_f,
        recv_b,
        sf_sems,
        rf_sems,
        sb_sems,
        rb_sems,
        credit_f,
        credit_b,
    ):
        my = lax.axis_index("i")
        left = lax.rem(my + N_DEV - 1, N_DEV)
        right = lax.rem(my + 1, N_DEV)

        barrier_sem = pltpu.get_barrier_semaphore()
        for nbr in (left, right):
            pl.semaphore_signal(
                barrier_sem, inc=1,
                device_id=(nbr,), device_id_type=pl.DeviceIdType.MESH,
            )
        pl.semaphore_wait(barrier_sem, 2)

        def top(c):
            return x_ref[0, 0:m_half, pl.ds(c * n_per, n_per)]

        def bot(c):
            return x_ref[0, m_half:m, pl.ds(c * n_per, n_per)]

        send_f[...] = top(left)
        send_b[...] = bot(right)

        for h in range(N_DEV - 1):
            slot = (h + 1) % 2
            if h >= 2:
                pl.semaphore_wait(credit_f, 1)
                pl.semaphore_wait(credit_b, 1)
            rf = pltpu.make_async_remote_copy(
                src_ref=send_f,
                dst_ref=recv_f.at[slot],
                send_sem=sf_sems.at[slot],
                recv_sem=rf_sems.at[slot],
                device_id=(right,),
                device_id_type=pl.DeviceIdType.MESH,
            )
            rb = pltpu.make_async_remote_copy(
                src_ref=send_b,
                dst_ref=recv_b.at[slot],
                send_sem=sb_sems.at[slot],
                recv_sem=rb_sems.at[slot],
                device_id=(left,),
                device_id_type=pl.DeviceIdType.MESH,
            )
            rf.start()
            rb.start()
            rf.wait()
            rb.wait()

            cf = lax.rem(my + 2 * N_DEV - h - 2, N_DEV)
            cb = lax.rem(my + h + 2, N_DEV)
            if h < N_DEV - 2:
                send_f[...] = recv_f[slot] + top(cf)
                send_b[...] = recv_b[slot] + bot(cb)
            else:
                out_ref[0:m_half, :] = recv_f[slot] + top(cf)
                out_ref[m_half:m, :] = recv_b[slot] + bot(cb)
            if h <= N_DEV - 4:
                pl.semaphore_signal(
                    credit_f, inc=1,
                    device_id=(left,), device_id_type=pl.DeviceIdType.MESH,
                )
                pl.semaphore_signal(
                    credit_b, inc=1,
                    device_id=(right,), device_id_type=pl.DeviceIdType.MESH,
                )

        @functools.partial(
            pl.run_scoped, second_barrier=pltpu.SemaphoreType.REGULAR
        )
        def _(second_barrier):
            for nbr in (left, right):
                pl.semaphore_signal(
                    second_barrier, inc=1,
                    device_id=(nbr,), device_id_type=pl.DeviceIdType.MESH,
                )
            pl.semaphore_wait(second_barrier, 2)

    return pl.pallas_call(
        body,
        out_shape=jax.ShapeDtypeStruct((m, n_per), x.dtype),
        in_specs=[pl.BlockSpec(memory_space=pltpu.VMEM)],
        out_specs=pl.BlockSpec(memory_space=pltpu.VMEM),
        scratch_shapes=[
            pltpu.VMEM((m_half, n_per), x.dtype),
            pltpu.VMEM((m_half, n_per), x.dtype),
            pltpu.VMEM((2, m_half, n_per), x.dtype),
            pltpu.VMEM((2, m_half, n_per), x.dtype),
            pltpu.SemaphoreType.DMA((2,)),
            pltpu.SemaphoreType.DMA((2,)),
            pltpu.SemaphoreType.DMA((2,)),
            pltpu.SemaphoreType.DMA((2,)),
            pltpu.SemaphoreType.REGULAR,
            pltpu.SemaphoreType.REGULAR,
        ],
        compiler_params=pltpu.CompilerParams(
            collective_id=0,
            vmem_limit_bytes=60 * 1024 * 1024,
        ),
    )(x)


# device time: 196202 ns/iter; 1.7962x vs baseline; 1.0096x over previous
import functools

import jax
import jax.numpy as jnp
from jax import lax
from jax.experimental import pallas as pl
from jax.experimental.pallas import tpu as pltpu

N_DEV = 8
SUBS = 2


def kernel(x):
    _, m, n_total = x.shape
    n_per = n_total // N_DEV
    m_half = m // 2
    sub_m = m_half // SUBS

    def body(
        x_ref,
        out_ref,
        send_f,
        send_b,
        recv_f,
        recv_b,
        sf_sems,
        rf_sems,
        sb_sems,
        rb_sems,
        credit_f,
        credit_b,
    ):
        my = lax.axis_index("i")
        left = lax.rem(my + N_DEV - 1, N_DEV)
        right = lax.rem(my + 1, N_DEV)

        def cols(c):
            return pl.ds(c * n_per, n_per)

        send_f[...] = x_ref[0, 0:m_half, cols(left)]
        send_b[...] = x_ref[0, m_half:m, cols(right)]

        barrier_sem = pltpu.get_barrier_semaphore()
        for nbr in (left, right):
            pl.semaphore_signal(
                barrier_sem, inc=1,
                device_id=(nbr,), device_id_type=pl.DeviceIdType.MESH,
            )
        pl.semaphore_wait(barrier_sem, 2)

        for h in range(N_DEV - 1):
            slot = (h + 1) % 2
            if h >= 2:
                pl.semaphore_wait(credit_f, 1)
                pl.semaphore_wait(credit_b, 1)

            def mk(src, dst, ssem, rsem, s, dev):
                return pltpu.make_async_remote_copy(
                    src_ref=src.at[pl.ds(s * sub_m, sub_m), :],
                    dst_ref=dst.at[slot, pl.ds(s * sub_m, sub_m), :],
                    send_sem=ssem.at[slot, s],
                    recv_sem=rsem.at[slot, s],
                    device_id=(dev,),
                    device_id_type=pl.DeviceIdType.MESH,
                )

            rf = [mk(send_f, recv_f, sf_sems, rf_sems, s, right) for s in range(SUBS)]
            rb = [mk(send_b, recv_b, sb_sems, rb_sems, s, left) for s in range(SUBS)]
            rf[0].start()
            rb[0].start()
            rf[1].start()
            rb[1].start()

            cf = lax.rem(my + 2 * N_DEV - h - 2, N_DEV)
            cb = lax.rem(my + h + 2, N_DEV)
            for s in range(SUBS):
                rows = pl.ds(s * sub_m, sub_m)
                xf_rows = pl.ds(s * sub_m, sub_m)
                xb_rows = pl.ds(m_half + s * sub_m, sub_m)
                rf[s].wait_send()
                rf[s].wait_recv()
                if h < N_DEV - 2:
                    send_f[rows, :] = recv_f[slot, rows, :] + x_ref[0, xf_rows, cols(cf)]
                else:
                    out_ref[xf_rows, :] = recv_f[slot, rows, :] + x_ref[0, xf_rows, cols(cf)]
                rb[s].wait_send()
                rb[s].wait_recv()
                if h < N_DEV - 2:
                    send_b[rows, :] = recv_b[slot, rows, :] + x_ref[0, xb_rows, cols(cb)]
                else:
                    out_ref[xb_rows, :] = recv_b[slot, rows, :] + x_ref[0, xb_rows, cols(cb)]

            if h <= N_DEV - 4:
                pl.semaphore_signal(
                    credit_f, inc=1,
                    device_id=(left,), device_id_type=pl.DeviceIdType.MESH,
                )
                pl.semaphore_signal(
                    credit_b, inc=1,
                    device_id=(right,), device_id_type=pl.DeviceIdType.MESH,
                )

        @functools.partial(
            pl.run_scoped, second_barrier=pltpu.SemaphoreType.REGULAR
        )
        def _(second_barrier):
            for nbr in (left, right):
                pl.semaphore_signal(
                    second_barrier, inc=1,
                    device_id=(nbr,), device_id_type=pl.DeviceIdType.MESH,
                )
            pl.semaphore_wait(second_barrier, 2)

    return pl.pallas_call(
        body,
        out_shape=jax.ShapeDtypeStruct((m, n_per), x.dtype),
        in_specs=[pl.BlockSpec(memory_space=pltpu.VMEM)],
        out_specs=pl.BlockSpec(memory_space=pltpu.VMEM),
        scratch_shapes=[
            pltpu.VMEM((m_half, n_per), x.dtype),
            pltpu.VMEM((m_half, n_per), x.dtype),
            pltpu.VMEM((2, m_half, n_per), x.dtype),
            pltpu.VMEM((2, m_half, n_per), x.dtype),
            pltpu.SemaphoreType.DMA((2, SUBS)),
            pltpu.SemaphoreType.DMA((2, SUBS)),
            pltpu.SemaphoreType.DMA((2, SUBS)),
            pltpu.SemaphoreType.DMA((2, SUBS)),
            pltpu.SemaphoreType.REGULAR,
            pltpu.SemaphoreType.REGULAR,
        ],
        compiler_params=pltpu.CompilerParams(
            collective_id=0,
            vmem_limit_bytes=60 * 1024 * 1024,
        ),
    )(x)


# device time: 186589 ns/iter; 1.8888x vs baseline; 1.0515x over previous
import functools

import jax
import jax.numpy as jnp
from jax import lax
from jax.experimental import pallas as pl
from jax.experimental.pallas import tpu as pltpu

N_DEV = 8
SUBS = 2


def kernel(x):
    _, m, n_total = x.shape
    n_per = n_total // N_DEV
    m_half = m // 2
    sub_m = m_half // SUBS

    def body(
        x_ref,
        out_ref,
        send_f,
        send_b,
        recv_f,
        recv_b,
        stage_f,
        stage_b,
        sf_sems,
        rf_sems,
        sb_sems,
        rb_sems,
        stf_sems,
        stb_sems,
        fill_sems,
        credit_f,
        credit_b,
    ):
        my = lax.axis_index("i")
        left = lax.rem(my + N_DEV - 1, N_DEV)
        right = lax.rem(my + 1, N_DEV)

        def cols(c):
            return pl.ds(c * n_per, n_per)

        def chunk_f(h):
            return lax.rem(my + 2 * N_DEV - h - 2, N_DEV)

        def chunk_b(h):
            return lax.rem(my + h + 2, N_DEV)

        def stage_hop(h):
            slot = h % 2
            f = pltpu.make_async_copy(
                x_ref.at[0, 0:m_half, cols(chunk_f(h))],
                stage_f.at[slot],
                stf_sems.at[slot],
            )
            b = pltpu.make_async_copy(
                x_ref.at[0, m_half:m, cols(chunk_b(h))],
                stage_b.at[slot],
                stb_sems.at[slot],
            )
            f.start()
            b.start()
            return f, b

        fill_f = pltpu.make_async_copy(
            x_ref.at[0, 0:m_half, cols(left)], send_f, fill_sems.at[0]
        )
        fill_b = pltpu.make_async_copy(
            x_ref.at[0, m_half:m, cols(right)], send_b, fill_sems.at[1]
        )
        fill_f.start()
        fill_b.start()
        stage = stage_hop(0)

        barrier_sem = pltpu.get_barrier_semaphore()
        for nbr in (left, right):
            pl.semaphore_signal(
                barrier_sem, inc=1,
                device_id=(nbr,), device_id_type=pl.DeviceIdType.MESH,
            )
        pl.semaphore_wait(barrier_sem, 2)
        fill_f.wait()
        fill_b.wait()

        for h in range(N_DEV - 1):
            slot = (h + 1) % 2
            if h >= 2:
                pl.semaphore_wait(credit_f, 1)
                pl.semaphore_wait(credit_b, 1)

            def mk(src, dst, ssem, rsem, s, dev):
                return pltpu.make_async_remote_copy(
                    src_ref=src.at[pl.ds(s * sub_m, sub_m), :],
                    dst_ref=dst.at[slot, pl.ds(s * sub_m, sub_m), :],
                    send_sem=ssem.at[slot, s],
                    recv_sem=rsem.at[slot, s],
                    device_id=(dev,),
                    device_id_type=pl.DeviceIdType.MESH,
                )

            rf = [mk(send_f, recv_f, sf_sems, rf_sems, s, right) for s in range(SUBS)]
            rb = [mk(send_b, recv_b, sb_sems, rb_sems, s, left) for s in range(SUBS)]
            rf[0].start()
            rb[0].start()
            rf[1].start()
            rb[1].start()

            if h + 1 < N_DEV - 1:
                next_stage = stage_hop(h + 1)

            st_slot = h % 2
            stage[0].wait()
            stage[1].wait()
            for s in range(SUBS):
                rows = pl.ds(s * sub_m, sub_m)
                out_rows_b = pl.ds(m_half + s * sub_m, sub_m)
                rf[s].wait_send()
                rf[s].wait_recv()
                if h < N_DEV - 2:
                    send_f[rows, :] = recv_f[slot, rows, :] + stage_f[st_slot, rows, :]
                else:
                    out_ref[rows, :] = recv_f[slot, rows, :] + stage_f[st_slot, rows, :]
                rb[s].wait_send()
                rb[s].wait_recv()
                if h < N_DEV - 2:
                    send_b[rows, :] = recv_b[slot, rows, :] + stage_b[st_slot, rows, :]
                else:
                    out_ref[out_rows_b, :] = recv_b[slot, rows, :] + stage_b[st_slot, rows, :]

            if h + 1 < N_DEV - 1:
                stage = next_stage
            if h <= N_DEV - 4:
                pl.semaphore_signal(
                    credit_f, inc=1,
                    device_id=(left,), device_id_type=pl.DeviceIdType.MESH,
                )
                pl.semaphore_signal(
                    credit_b, inc=1,
                    device_id=(right,), device_id_type=pl.DeviceIdType.MESH,
                )

        @functools.partial(
            pl.run_scoped, second_barrier=pltpu.SemaphoreType.REGULAR
        )
        def _(second_barrier):
            for nbr in (left, right):
                pl.semaphore_signal(
                    second_barrier, inc=1,
                    device_id=(nbr,), device_id_type=pl.DeviceIdType.MESH,
                )
            pl.semaphore_wait(second_barrier, 2)

    return pl.pallas_call(
        body,
        out_shape=jax.ShapeDtypeStruct((m, n_per), x.dtype),
        in_specs=[pl.BlockSpec(memory_space=pl.ANY)],
        out_specs=pl.BlockSpec(memory_space=pltpu.VMEM),
        scratch_shapes=[
            pltpu.VMEM((m_half, n_per), x.dtype),
            pltpu.VMEM((m_half, n_per), x.dtype),
            pltpu.VMEM((2, m_half, n_per), x.dtype),
            pltpu.VMEM((2, m_half, n_per), x.dtype),
            pltpu.VMEM((2, m_half, n_per), x.dtype),
            pltpu.VMEM((2, m_half, n_per), x.dtype),
            pltpu.SemaphoreType.DMA((2, SUBS)),
            pltpu.SemaphoreType.DMA((2, SUBS)),
            pltpu.SemaphoreType.DMA((2, SUBS)),
            pltpu.SemaphoreType.DMA((2, SUBS)),
            pltpu.SemaphoreType.DMA((2,)),
            pltpu.SemaphoreType.DMA((2,)),
            pltpu.SemaphoreType.DMA((2,)),
            pltpu.SemaphoreType.REGULAR,
            pltpu.SemaphoreType.REGULAR,
        ],
        compiler_params=pltpu.CompilerParams(
            collective_id=0,
            vmem_limit_bytes=60 * 1024 * 1024,
        ),
    )(x)


# device time: 172860 ns/iter; 2.0388x vs baseline; 1.0794x over previous
import functools

import jax
import jax.numpy as jnp
from jax import lax
from jax.experimental import pallas as pl
from jax.experimental.pallas import tpu as pltpu

N_DEV = 8
N_HOP = N_DEV - 1
SUBS = 2


def kernel(x):
    _, m, n_total = x.shape
    n_per = n_total // N_DEV
    m_half = m // 2
    sub_m = m_half // SUBS

    def body(
        x_ref,
        out_ref,
        send_f,
        send_b,
        recv_f,
        recv_b,
        stage_f,
        stage_b,
        sf_sems,
        rf_sems,
        sb_sems,
        rb_sems,
        stf_sems,
        stb_sems,
        fill_sems,
        credit_f,
        credit_b,
    ):
        my = lax.axis_index("i")
        left = lax.rem(my + N_DEV - 1, N_DEV)
        right = lax.rem(my + 1, N_DEV)

        def cols(c):
            return pl.ds(c * n_per, n_per)

        def chunk_f(h):
            return lax.rem(my + 2 * N_DEV - h - 2, N_DEV)

        def chunk_b(h):
            return lax.rem(my + h + 2, N_DEV)

        def stage_hop(h):
            slot = h % 2
            f = pltpu.make_async_copy(
                x_ref.at[0, 0:m_half, cols(chunk_f(h))],
                stage_f.at[slot],
                stf_sems.at[slot],
            )
            b = pltpu.make_async_copy(
                x_ref.at[0, m_half:m, cols(chunk_b(h))],
                stage_b.at[slot],
                stb_sems.at[slot],
            )
            f.start()
            b.start()
            return f, b

        def mk(h, s, fwd):
            slot = (h + 1) % 2
            src = send_f if fwd else send_b
            dst = recv_f if fwd else recv_b
            ssem = sf_sems if fwd else sb_sems
            rsem = rf_sems if fwd else rb_sems
            return pltpu.make_async_remote_copy(
                src_ref=src.at[pl.ds(s * sub_m, sub_m), :],
                dst_ref=dst.at[slot, pl.ds(s * sub_m, sub_m), :],
                send_sem=ssem.at[slot, s],
                recv_sem=rsem.at[slot, s],
                device_id=(right if fwd else left,),
                device_id_type=pl.DeviceIdType.MESH,
            )

        fill_f = pltpu.make_async_copy(
            x_ref.at[0, 0:m_half, cols(left)], send_f, fill_sems.at[0]
        )
        fill_b = pltpu.make_async_copy(
            x_ref.at[0, m_half:m, cols(right)], send_b, fill_sems.at[1]
        )
        fill_f.start()
        fill_b.start()
        stage = stage_hop(0)

        barrier_sem = pltpu.get_barrier_semaphore()
        for nbr in (left, right):
            pl.semaphore_signal(
                barrier_sem, inc=1,
                device_id=(nbr,), device_id_type=pl.DeviceIdType.MESH,
            )
        pl.semaphore_wait(barrier_sem, 2)
        fill_f.wait()
        fill_b.wait()

        inflight = {}
        for s in range(SUBS):
            inflight[(0, s, True)] = mk(0, s, True)
            inflight[(0, s, False)] = mk(0, s, False)
            inflight[(0, s, True)].start()
            inflight[(0, s, False)].start()

        for h in range(N_HOP):
            slot = (h + 1) % 2
            st_slot = h % 2
            if h + 1 < N_HOP:
                next_stage = stage_hop(h + 1)
            stage[0].wait()
            stage[1].wait()

            for s in range(SUBS):
                rows = pl.ds(s * sub_m, sub_m)
                out_rows_b = pl.ds(m_half + s * sub_m, sub_m)
                for fwd in (True, False):
                    rdma = inflight.pop((h, s, fwd))
                    rdma.wait_send()
                    rdma.wait_recv()
                    recv = recv_f if fwd else recv_b
                    st = stage_f if fwd else stage_b
                    if h < N_HOP - 1:
                        dst = send_f if fwd else send_b
                        dst[rows, :] = recv[slot, rows, :] + st[st_slot, rows, :]
                    else:
                        orow = rows if fwd else out_rows_b
                        out_ref[orow, :] = recv[slot, rows, :] + st[st_slot, rows, :]
                    if h <= N_HOP - 3:
                        pl.semaphore_signal(
                            (credit_f if fwd else credit_b).at[s],
                            inc=1,
                            device_id=(left if fwd else right,),
                            device_id_type=pl.DeviceIdType.MESH,
                        )
                    if h + 1 < N_HOP:
                        if h + 1 >= 2:
                            pl.semaphore_wait(
                                (credit_f if fwd else credit_b).at[s], 1
                            )
                        nxt = mk(h + 1, s, fwd)
                        inflight[(h + 1, s, fwd)] = nxt
                        nxt.start()

            if h + 1 < N_HOP:
                stage = next_stage

        @functools.partial(
            pl.run_scoped, second_barrier=pltpu.SemaphoreType.REGULAR
        )
        def _(second_barrier):
            for nbr in (left, right):
                pl.semaphore_signal(
                    second_barrier, inc=1,
                    device_id=(nbr,), device_id_type=pl.DeviceIdType.MESH,
                )
            pl.semaphore_wait(second_barrier, 2)

    return pl.pallas_call(
        body,
        out_shape=jax.ShapeDtypeStruct((m, n_per), x.dtype),
        in_specs=[pl.BlockSpec(memory_space=pl.ANY)],
        out_specs=pl.BlockSpec(memory_space=pltpu.VMEM),
        scratch_shapes=[
            pltpu.VMEM((m_half, n_per), x.dtype),
            pltpu.VMEM((m_half, n_per), x.dtype),
            pltpu.VMEM((2, m_half, n_per), x.dtype),
            pltpu.VMEM((2, m_half, n_per), x.dtype),
            pltpu.VMEM((2, m_half, n_per), x.dtype),
            pltpu.VMEM((2, m_half, n_per), x.dtype),
            pltpu.SemaphoreType.DMA((2, SUBS)),
            pltpu.SemaphoreType.DMA((2, SUBS)),
            pltpu.SemaphoreType.DMA((2, SUBS)),
            pltpu.SemaphoreType.DMA((2, SUBS)),
            pltpu.SemaphoreType.DMA((2,)),
            pltpu.SemaphoreType.DMA((2,)),
            pltpu.SemaphoreType.DMA((2,)),
            pltpu.SemaphoreType.REGULAR((SUBS,)),
            pltpu.SemaphoreType.REGULAR((SUBS,)),
        ],
        compiler_params=pltpu.CompilerParams(
            collective_id=0,
            vmem_limit_bytes=60 * 1024 * 1024,
        ),
    )(x)
